# baseline (speedup 1.0000x reference)
import dataclasses
import functools

import jax
import jax.numpy as jnp
from jax import lax
from jax.experimental import pallas as pl
from jax.experimental.pallas import tpu as pltpu
from jax.experimental.pallas import tpu_sc as plsc

N = 10000
E = 320000
D = 128

NW = 32
G = 128
H = G // 2
NE = E + N
GROUPS = 2 * (-(-NE // (NW * G * 2)))
TOTAL = NW * GROUPS * G
NT = 10112
NA = 10112
JUNK = N
RPT = NA // 16


def _loop(n):
    return pl.loop(jnp.int32(0), jnp.int32(n))


def _tc_prep(x, W, att2):
    def body(x_ref, w_ref, a_ref, h_ref, aux_ref):
        h = jnp.dot(x_ref[...], w_ref[...],
                    preferred_element_type=jnp.float32,
                    precision=lax.Precision.HIGHEST)
        h_ref[...] = h
        aux = lax.dot_general(a_ref[...], h, (((1,), (1,)), ((), ())),
                              preferred_element_type=jnp.float32,
                              precision=lax.Precision.HIGHEST)
        aux_ref[...] = jnp.concatenate(
            [aux, jnp.zeros((2, NT - N), jnp.float32)], axis=1)

    return pl.pallas_call(
        body,
        out_shape=(jax.ShapeDtypeStruct((N, D), jnp.float32),
                   jax.ShapeDtypeStruct((2, NT), jnp.float32)),
    )(x, W, att2)


def _sc_edges(h, aux, idx_t):
    mesh = plsc.VectorSubcoreMesh(core_axis_name="c", subcore_axis_name="s",
                                  num_cores=2, num_subcores=16)
    cp = pltpu.CompilerParams()
    if "needs_layout_passes" in pltpu.CompilerParams.__dataclass_fields__:
        cp = dataclasses.replace(cp, needs_layout_passes=False)

    @functools.partial(
        pl.kernel,
        out_type=(jax.ShapeDtypeStruct((2, NA, D), jnp.float32),
                  jax.ShapeDtypeStruct((NW, NA), jnp.float32)),
        mesh=mesh,
        scratch_types=[
            pltpu.VMEM((NT,), jnp.float32),
            pltpu.VMEM((NT,), jnp.float32),
            pltpu.VMEM((2, 4, H), jnp.int32),
            pltpu.VMEM((NA,), jnp.float32),
            pltpu.VMEM((2, H, D), jnp.float32),
            pltpu.VMEM((H,), jnp.float32),
            pltpu.VMEM_SHARED((NA, D), jnp.float32),
            pltpu.SemaphoreType.DMA,
            pltpu.SemaphoreType.DMA,
            pltpu.SemaphoreType.DMA,
        ],
        compiler_params=cp,
    )
    def k(h_hbm, aux_hbm, idx_hbm, acc_hbm, den_hbm,
          asrc_v, adst_v, ibuf, den_v, rows_v, p_v, acc_sh,
          sem_a, sem_b, sem_i):
        i32 = jnp.int32
        c = lax.axis_index("c").astype(i32)
        s = lax.axis_index("s").astype(i32)
        wid = c * i32(16) + s
        zv = jnp.zeros((16,), jnp.float32)
        sem_r = (sem_a, sem_b)

        @_loop(NA // 16)
        def _(i):
            den_v[pl.ds(pl.multiple_of(i * i32(16), 8), 16)] = zv

        @_loop(H)
        def _(r):
            for b in range(2):
                for j in range(8):
                    rows_v[b, r, pl.ds(j * 16, 16)] = zv

        @_loop(RPT // H)
        def _(i):
            pltpu.sync_copy(
                rows_v.at[i32(0)],
                acc_sh.at[pl.ds(s * i32(RPT) + i * i32(H), H)])
        rem = RPT - (RPT // H) * H
        if rem:
            pltpu.sync_copy(
                rows_v.at[i32(0), pl.ds(0, rem)],
                acc_sh.at[pl.ds(s * i32(RPT) + i32(RPT - rem), rem)])

        pltpu.sync_copy(aux_hbm.at[i32(0)], asrc_v)
        pltpu.sync_copy(aux_hbm.at[i32(1)], adst_v)
        plsc.subcore_barrier()

        def attn(pb, half):
            for j in range(H // 16):
                sv = ibuf[pb, half, pl.ds(j * 16, 16)]
                dv = ibuf[pb, 2 + half, pl.ds(j * 16, 16)]
                e = (plsc.load_gather(asrc_v, [sv])
                     + plsc.load_gather(adst_v, [dv]))
                e = jnp.where(e >= 0.0, e, 0.2 * e)
                p = jnp.exp(e)
                plsc.addupdate_scatter(den_v, [dv], p)
                p_v[pl.ds(j * 16, 16)] = p

        def scale(b):
            @_loop(H // 4)
            def _(eb):
                e4 = eb * i32(4)
                for u in range(4):
                    av = plsc.load_gather(
                        p_v, [jnp.full((16,), e4 + i32(u), i32)])
                    for j in range(8):
                        r = rows_v[b, e4 + i32(u), pl.ds(j * 16, 16)]
                        rows_v[b, e4 + i32(u), pl.ds(j * 16, 16)] = r * av

        def fire_rows(pb, half, b):
            pltpu.async_copy(h_hbm.at[ibuf.at[i32(pb), i32(half)]],
                             rows_v.at[i32(b)], sem_r[b])

        def wait_rows(b):
            pltpu.make_async_copy(h_hbm.at[pl.ds(i32(0), H)],
                                  rows_v.at[i32(b)], sem_r[b]).wait()

        def scatter(pb, half, b):
            pltpu.sync_copy(rows_v.at[i32(b)],
                            acc_sh.at[ibuf.at[i32(pb), i32(2 + half)]],
                            add=True)

        pltpu.sync_copy(idx_hbm.at[wid, i32(0)], ibuf.at[i32(0)])
        pltpu.async_copy(idx_hbm.at[wid, i32(1)], ibuf.at[i32(1)], sem_i)
        fire_rows(0, 0, 0)

        QL = GROUPS // 2

        @_loop(QL)
        def _(q):
            last = q >= i32(QL - 1)
            for pb in range(2):
                g = q * i32(2) + i32(pb)
                fire_rows(pb, 1, 1)
                attn(pb, 0)
                wait_rows(0)
                scale(0)
                scatter(pb, 0, 0)
                attn(pb, 1)

                def _wait_i_fire_next(pb=pb):
                    pltpu.make_async_copy(idx_hbm.at[wid, i32(0)],
                                          ibuf.at[i32(1 - pb)],
                                          sem_i).wait()
                    fire_rows(1 - pb, 0, 0)
                if pb == 0:
                    _wait_i_fire_next()
                else:
                    pl.when(jnp.logical_not(last))(_wait_i_fire_next)

                wait_rows(1)
                scale(1)
                scatter(pb, 1, 1)

                @pl.when(jnp.logical_not(last))
                def _():
                    pltpu.async_copy(idx_hbm.at[wid, g + i32(2)],
                                     ibuf.at[i32(pb)], sem_i)

        plsc.subcore_barrier()
        pltpu.sync_copy(acc_sh.at[pl.ds(s * i32(RPT), RPT)],
                        acc_hbm.at[c, pl.ds(s * i32(RPT), RPT)])
        pltpu.sync_copy(den_v, den_hbm.at[wid])

    return k(h, aux, idx_t)


def _tc_final(acc, den, bias2):
    def body(acc_ref, den_ref, b_ref, o_ref):
        a = acc_ref[0] + acc_ref[1]
        dsum = jnp.sum(den_ref[...], axis=0)
        o_ref[...] = jnp.tanh(
            a[:N] / (dsum[:N, None] + 1e-16) + b_ref[...])

    return pl.pallas_call(
        body,
        out_shape=jax.ShapeDtypeStruct((N, D), jnp.float32),
    )(acc, den, bias2)


def kernel(x, edge_index, W, att_src, att_dst, bias):
    src = edge_index[0].astype(jnp.int32)
    dst = edge_index[1].astype(jnp.int32)
    loop = jnp.arange(N, dtype=jnp.int32)
    pad = TOTAL - NE
    src_all = jnp.concatenate(
        [src, loop, jnp.zeros((pad,), jnp.int32)])
    dst_all = jnp.concatenate(
        [dst, loop, jnp.full((pad,), JUNK, jnp.int32)])
    idx_t = jnp.concatenate(
        [src_all.reshape(NW, GROUPS, 2, H),
         dst_all.reshape(NW, GROUPS, 2, H)], axis=2)
    att2 = jnp.stack([att_src, att_dst]).astype(jnp.float32)

    h, aux = _tc_prep(x.astype(jnp.float32), W.astype(jnp.float32), att2)
    acc, den = _sc_edges(h, aux, idx_t)
    out = _tc_final(acc, den, bias.astype(jnp.float32).reshape(1, D))
    return out.astype(jnp.result_type(x.dtype, W.dtype))
